# baseline (speedup 1.0000x reference)
import functools

import jax
import jax.numpy as jnp
import numpy as np
from jax import lax
from jax.experimental import pallas as pl
from jax.experimental.pallas import tpu as pltpu
from jax.experimental.pallas import tpu_sc as plsc

N = 10000
E = 320000
D = 128
DH = D // 2

NC = 2
NS = 16
CB = 128
CH = 160
PADE = NS * CH * CB
NCHK = PADE // CB
NP = 10240
ROWS_PER_TILE = NP // NS
NBUF = 4
NGRP = CH // NBUF
SB = 16
NSB = N // SB

WBLK = 256


def _wexp_tc(w2d):
    def body(w_ref, o_ref):
        for t in range(16):
            pieces = [
                jnp.broadcast_to(
                    w_ref[:, 8 * t + q:8 * t + q + 1], (WBLK, 16))
                for q in range(8)
            ]
            o_ref[:, t, :] = jnp.concatenate(pieces, axis=1)

    return pl.pallas_call(
        body,
        grid=(NCHK // WBLK,),
        in_specs=[
            pl.BlockSpec((WBLK, 128), lambda i: (i, 0)),
        ],
        out_specs=pl.BlockSpec((WBLK, 16, 128), lambda i: (i, 0, 0)),
        out_shape=jax.ShapeDtypeStruct((NCHK, 16, 128), jnp.float32),
    )(w2d)


def _spmm_sc(packed, wexp3, x):
    mesh = plsc.VectorSubcoreMesh(core_axis_name="c", subcore_axis_name="s")

    @functools.partial(
        pl.kernel,
        out_type=jax.ShapeDtypeStruct((NC, NP, DH), jnp.float32),
        mesh=mesh,
        scratch_types=[
            [pltpu.VMEM((CB,), jnp.int32) for _ in range(NBUF)],
            [pltpu.VMEM((CB,), jnp.int32) for _ in range(NBUF)],
            [pltpu.VMEM((CB,), jnp.int32) for _ in range(NBUF)],
            [pltpu.VMEM((16, 128), jnp.float32) for _ in range(2)],
            [pltpu.VMEM((CB, DH), jnp.float32) for _ in range(NBUF)],
            pltpu.VMEM((SB, D), jnp.float32),
            pltpu.VMEM((SB, DH), jnp.float32),
            pltpu.VMEM_SHARED((NP, DH), jnp.float32),
            pltpu.VMEM_SHARED((NP, DH), jnp.float32),
            [pltpu.SemaphoreType.DMA for _ in range(NBUF)],
            [pltpu.SemaphoreType.DMA for _ in range(NBUF)],
            [pltpu.SemaphoreType.DMA for _ in range(NBUF)],
            [pltpu.SemaphoreType.DMA for _ in range(2)],
        ],
        compiler_params=pltpu.CompilerParams(use_tc_tiling_on_sc=False),
    )
    def spmm(packed_hbm, wexp_hbm, x_hbm, out_hbm,
             packed_v, src_v, dst_v, wexp_v, rows_v, xtmp, xhalf, xsh, acc,
             psem, gsem, ssem, wsem):
        cid = lax.axis_index("c")
        sid = lax.axis_index("s")
        r0s = sid * ROWS_PER_TILE

        def stage_block(k, _):
            g = sid * (ROWS_PER_TILE // SB) + k

            @pl.when(g < NSB)
            def _():
                pltpu.sync_copy(x_hbm.at[pl.ds(g * SB, SB)], xtmp)
                for r in range(SB):
                    for j in range(DH // 16):
                        xhalf[r, pl.ds(j * 16, 16)] = (
                            xtmp[r, pl.ds(cid * DH + j * 16, 16)])
                pltpu.sync_copy(xhalf, xsh.at[pl.ds(g * SB, SB)])
            return 0

        lax.fori_loop(0, ROWS_PER_TILE // SB, stage_block, 0)

        zvec = jnp.zeros((16,), jnp.float32)

        def zero_row(r, _):
            for j in range(DH // 16):
                rows_v[0][r, pl.ds(j * 16, 16)] = zvec
            return 0

        lax.fori_loop(0, CB, zero_row, 0)
        for k in range(ROWS_PER_TILE // CB):
            pltpu.sync_copy(rows_v[0], acc.at[pl.ds(r0s + k * CB, CB)])
        plsc.subcore_barrier()

        def start_packed(i, p):
            pltpu.async_copy(
                packed_hbm.at[pl.ds((sid * CH + i) * CB, CB)],
                packed_v[p], psem[p])

        def wait_packed(i, p):
            pltpu.make_async_copy(
                packed_hbm.at[pl.ds((sid * CH + i) * CB, CB)],
                packed_v[p], psem[p]).wait()

        def unpack_src(p):
            for g in range(CB // 16):
                v = packed_v[p][pl.ds(g * 16, 16)]
                src_v[p][pl.ds(g * 16, 16)] = v & 0xFFFF

        def unpack_dst(p):
            for g in range(CB // 16):
                v = packed_v[p][pl.ds(g * 16, 16)]
                dst_v[p][pl.ds(g * 16, 16)] = lax.shift_right_logical(v, 16)

        def start_fetch(i, b):
            pltpu.async_copy(xsh.at[src_v[b]], rows_v[b], gsem[b])
            pltpu.async_copy(
                wexp_hbm.at[sid * CH + i], wexp_v[b % 2], wsem[b % 2])

        def wait_fetch(i, b):
            pltpu.make_async_copy(
                xsh.at[src_v[b]], rows_v[b], gsem[b]).wait()
            pltpu.make_async_copy(
                wexp_hbm.at[sid * CH + i], wexp_v[b % 2], wsem[b % 2]).wait()

        def start_scatter(i, b):
            pltpu.async_copy(
                rows_v[b], acc.at[dst_v[b]], ssem[b], add=True)

        def wait_scatter(i, b):
            pltpu.make_async_copy(
                rows_v[b], acc.at[dst_v[b]], ssem[b]).wait()

        for p in range(NBUF):
            start_packed(p, p)
        for b in range(2):
            wait_packed(b, b)
            unpack_src(b)
            start_fetch(b, b)

        def group_body(g, _):
            for b in range(NBUF):
                i = g * NBUF + b
                wait_fetch(i, b)

                def scale_body(t, _, b=b):
                    for kk in range(8):
                        wb = wexp_v[b % 2][t, pl.ds(kk * 16, 16)]
                        for j in range(DH // 16):
                            sl = pl.ds(j * 16, 16)
                            rows_v[b][t * 8 + kk, sl] = (
                                rows_v[b][t * 8 + kk, sl] * wb)
                    return 0

                lax.fori_loop(0, CB // 8, scale_body, 0)
                unpack_dst(b)
                start_scatter(i, b)

                j2 = i + 2
                b2 = (b + 2) % NBUF

                @pl.when(j2 < CH)
                def _():
                    wait_packed(j2, b2)
                    unpack_src(b2)

                    @pl.when(j2 >= NBUF)
                    def _():
                        wait_scatter(j2 - NBUF, b2)
                    start_fetch(j2, b2)

                j4 = i + 4

                @pl.when(j4 < CH)
                def _():
                    start_packed(j4, b)
            return 0

        lax.fori_loop(0, NGRP, group_body, 0)
        for k in range(CH - NBUF, CH):
            wait_scatter(k, k % NBUF)
        plsc.subcore_barrier()

        for k in range(ROWS_PER_TILE // CB):
            r0 = r0s + k * CB
            pltpu.sync_copy(acc.at[pl.ds(r0, CB)],
                            out_hbm.at[cid, pl.ds(r0, CB)])

    return spmm(packed, wexp3, x)


BLK = 1000


def _linear_tc(halves, W, b2d):
    def body(p_ref, w_ref, b_ref, o_ref):
        s = jnp.concatenate([p_ref[0], p_ref[1]], axis=1)
        o_ref[...] = jnp.dot(
            s, w_ref[...], preferred_element_type=jnp.float32) + b_ref[...]

    return pl.pallas_call(
        body,
        grid=(N // BLK,),
        in_specs=[
            pl.BlockSpec((NC, BLK, DH), lambda i: (0, i, 0)),
            pl.BlockSpec((D, D), lambda i: (0, 0)),
            pl.BlockSpec((1, D), lambda i: (0, 0)),
        ],
        out_specs=pl.BlockSpec((BLK, D), lambda i: (i, 0)),
        out_shape=jax.ShapeDtypeStruct((N, D), jnp.float32),
    )(halves, W, b2d)


@jax.jit
def kernel(x, edge_index, edge_weight, W, b):
    dst = edge_index[0]
    src = edge_index[1]
    npad = PADE - E
    parange = jnp.arange(npad, dtype=jnp.int32)
    src_pad = jnp.concatenate([src, parange % N])
    dst_pad = jnp.concatenate([dst, N + parange % (NP - N)])
    packed = jnp.left_shift(dst_pad, 16) | src_pad
    wpad = jnp.concatenate([edge_weight, jnp.zeros((npad,), jnp.float32)])
    wexp3 = _wexp_tc(wpad.reshape(NCHK, 128))
    halves = _spmm_sc(packed, wexp3, x)
    return _linear_tc(halves, W, b.reshape(1, D))


# speedup vs baseline: 1.0950x; 1.0950x over previous
import functools

import jax
import jax.numpy as jnp
import numpy as np
from jax import lax
from jax.experimental import pallas as pl
from jax.experimental.pallas import tpu as pltpu
from jax.experimental.pallas import tpu_sc as plsc

N = 10000
E = 320000
D = 128
DH = D // 2

NC = 2
NS = 16
CB = 128
CH = 160
PADE = NS * CH * CB
NCHK = PADE // CB
NP = 10240
ROWS_PER_TILE = NP // NS
NBUF = 4
NGRP = CH // NBUF
SB = 16
NSB = N // SB

_sel8 = np.zeros((8, 128), np.float32)
for _c in range(128):
    _sel8[_c // 16, _c] = 1.0

WBLK = 256


def _wexp_tc(w2d):
    def body(w_ref, s_ref, o_ref):
        for t in range(16):
            o_ref[:, t, :] = jnp.dot(
                w_ref[:, 8 * t:8 * t + 8], s_ref[...],
                preferred_element_type=jnp.float32,
                precision=lax.Precision.HIGHEST)

    return pl.pallas_call(
        body,
        grid=(NCHK // WBLK,),
        in_specs=[
            pl.BlockSpec((WBLK, 128), lambda i: (i, 0)),
            pl.BlockSpec((8, 128), lambda i: (0, 0)),
        ],
        out_specs=pl.BlockSpec((WBLK, 16, 128), lambda i: (i, 0, 0)),
        out_shape=jax.ShapeDtypeStruct((NCHK, 16, 128), jnp.float32),
    )(w2d, jnp.asarray(_sel8))


def _spmm_sc(packed, wexp3, x):
    mesh = plsc.VectorSubcoreMesh(core_axis_name="c", subcore_axis_name="s")

    @functools.partial(
        pl.kernel,
        out_type=jax.ShapeDtypeStruct((NC, NP, DH), jnp.float32),
        mesh=mesh,
        scratch_types=[
            [pltpu.VMEM((CB,), jnp.int32) for _ in range(NBUF)],
            [pltpu.VMEM((CB,), jnp.int32) for _ in range(NBUF)],
            [pltpu.VMEM((CB,), jnp.int32) for _ in range(NBUF)],
            [pltpu.VMEM((16, 128), jnp.float32) for _ in range(2)],
            [pltpu.VMEM((CB, DH), jnp.float32) for _ in range(NBUF)],
            pltpu.VMEM((SB, D), jnp.float32),
            pltpu.VMEM((SB, DH), jnp.float32),
            pltpu.VMEM_SHARED((NP, DH), jnp.float32),
            pltpu.VMEM_SHARED((NP, DH), jnp.float32),
            [pltpu.SemaphoreType.DMA for _ in range(NBUF)],
            [pltpu.SemaphoreType.DMA for _ in range(NBUF)],
            [pltpu.SemaphoreType.DMA for _ in range(NBUF)],
            [pltpu.SemaphoreType.DMA for _ in range(2)],
        ],
        compiler_params=pltpu.CompilerParams(use_tc_tiling_on_sc=False),
    )
    def spmm(packed_hbm, wexp_hbm, x_hbm, out_hbm,
             packed_v, src_v, dst_v, wexp_v, rows_v, xtmp, xhalf, xsh, acc,
             psem, gsem, ssem, wsem):
        cid = lax.axis_index("c")
        sid = lax.axis_index("s")
        r0s = sid * ROWS_PER_TILE

        def stage_block(k, _):
            g = sid * (ROWS_PER_TILE // SB) + k

            @pl.when(g < NSB)
            def _():
                pltpu.sync_copy(x_hbm.at[pl.ds(g * SB, SB)], xtmp)
                for r in range(SB):
                    for j in range(DH // 16):
                        xhalf[r, pl.ds(j * 16, 16)] = (
                            xtmp[r, pl.ds(cid * DH + j * 16, 16)])
                pltpu.sync_copy(xhalf, xsh.at[pl.ds(g * SB, SB)])
            return 0

        lax.fori_loop(0, ROWS_PER_TILE // SB, stage_block, 0)

        zvec = jnp.zeros((16,), jnp.float32)

        def zero_row(r, _):
            for j in range(DH // 16):
                rows_v[0][r, pl.ds(j * 16, 16)] = zvec
            return 0

        lax.fori_loop(0, CB, zero_row, 0)
        for k in range(ROWS_PER_TILE // CB):
            pltpu.sync_copy(rows_v[0], acc.at[pl.ds(r0s + k * CB, CB)])
        plsc.subcore_barrier()

        def start_packed(i, p):
            pltpu.async_copy(
                packed_hbm.at[pl.ds((sid * CH + i) * CB, CB)],
                packed_v[p], psem[p])

        def wait_packed(i, p):
            pltpu.make_async_copy(
                packed_hbm.at[pl.ds((sid * CH + i) * CB, CB)],
                packed_v[p], psem[p]).wait()

        def unpack_src(p):
            for g in range(CB // 16):
                v = packed_v[p][pl.ds(g * 16, 16)]
                src_v[p][pl.ds(g * 16, 16)] = v & 0xFFFF

        def unpack_dst(p):
            for g in range(CB // 16):
                v = packed_v[p][pl.ds(g * 16, 16)]
                dst_v[p][pl.ds(g * 16, 16)] = lax.shift_right_logical(v, 16)

        def start_fetch(i, b):
            pltpu.async_copy(xsh.at[src_v[b]], rows_v[b], gsem[b])
            pltpu.async_copy(
                wexp_hbm.at[sid * CH + i], wexp_v[b % 2], wsem[b % 2])

        def wait_fetch(i, b):
            pltpu.make_async_copy(
                xsh.at[src_v[b]], rows_v[b], gsem[b]).wait()
            pltpu.make_async_copy(
                wexp_hbm.at[sid * CH + i], wexp_v[b % 2], wsem[b % 2]).wait()

        def start_scatter(i, b):
            pltpu.async_copy(
                rows_v[b], acc.at[dst_v[b]], ssem[b], add=True)

        def wait_scatter(i, b):
            pltpu.make_async_copy(
                rows_v[b], acc.at[dst_v[b]], ssem[b]).wait()

        for p in range(NBUF):
            start_packed(p, p)
        for b in range(2):
            wait_packed(b, b)
            unpack_src(b)
            start_fetch(b, b)

        def group_body(g, _):
            for b in range(NBUF):
                i = g * NBUF + b
                wait_fetch(i, b)

                def scale_body(t, _, b=b):
                    for kk in range(8):
                        wb = wexp_v[b % 2][t, pl.ds(kk * 16, 16)]
                        for j in range(DH // 16):
                            sl = pl.ds(j * 16, 16)
                            rows_v[b][t * 8 + kk, sl] = (
                                rows_v[b][t * 8 + kk, sl] * wb)
                    return 0

                lax.fori_loop(0, CB // 8, scale_body, 0)
                unpack_dst(b)
                start_scatter(i, b)

                j2 = i + 2
                b2 = (b + 2) % NBUF

                @pl.when(j2 < CH)
                def _():
                    wait_packed(j2, b2)
                    unpack_src(b2)

                    @pl.when(j2 >= NBUF)
                    def _():
                        wait_scatter(j2 - NBUF, b2)
                    start_fetch(j2, b2)

                j4 = i + 4

                @pl.when(j4 < CH)
                def _():
                    start_packed(j4, b)
            return 0

        lax.fori_loop(0, NGRP, group_body, 0)
        for k in range(CH - NBUF, CH):
            wait_scatter(k, k % NBUF)
        plsc.subcore_barrier()

        for k in range(ROWS_PER_TILE // CB):
            r0 = r0s + k * CB
            pltpu.sync_copy(acc.at[pl.ds(r0, CB)],
                            out_hbm.at[cid, pl.ds(r0, CB)])

    return spmm(packed, wexp3, x)


BLK = 1000


def _linear_tc(halves, W, b2d):
    def body(p_ref, w_ref, b_ref, o_ref):
        s = jnp.concatenate([p_ref[0], p_ref[1]], axis=1)
        o_ref[...] = jnp.dot(
            s, w_ref[...], preferred_element_type=jnp.float32) + b_ref[...]

    return pl.pallas_call(
        body,
        grid=(N // BLK,),
        in_specs=[
            pl.BlockSpec((NC, BLK, DH), lambda i: (0, i, 0)),
            pl.BlockSpec((D, D), lambda i: (0, 0)),
            pl.BlockSpec((1, D), lambda i: (0, 0)),
        ],
        out_specs=pl.BlockSpec((BLK, D), lambda i: (i, 0)),
        out_shape=jax.ShapeDtypeStruct((N, D), jnp.float32),
    )(halves, W, b2d)


@jax.jit
def kernel(x, edge_index, edge_weight, W, b):
    dst = edge_index[0]
    src = edge_index[1]
    npad = PADE - E
    parange = jnp.arange(npad, dtype=jnp.int32)
    src_pad = jnp.concatenate([src, parange % N])
    dst_pad = jnp.concatenate([dst, N + parange % (NP - N)])
    packed = jnp.left_shift(dst_pad, 16) | src_pad
    wpad = jnp.concatenate([edge_weight, jnp.zeros((npad,), jnp.float32)])
    wexp3 = _wexp_tc(wpad.reshape(NCHK, 128))
    halves = _spmm_sc(packed, wexp3, x)
    return _linear_tc(halves, W, b.reshape(1, D))
